# baseline (speedup 1.0000x reference)
import jax
import jax.numpy as jnp
from jax import lax
from jax.experimental import pallas as pl
from jax.experimental.pallas import tpu as pltpu
from jax.experimental.pallas import tpu_sc as plsc

_B = 16384
_D = 128
_NC = 2
_NS = 16
_NW = _NC * _NS
_BPW = _B // _NW
_CH = 128
_NCHUNK = _BPW // _CH
_NBUF = 2


def _gmf_body(users_hbm, items_hbm, utab_hbm, itab_hbm, w_hbm, b_hbm,
              out_hbm, uidx_v, iidx_v, ub0, ub1, ib0, ib1,
              w_v, binit_v, qbuf_v, outc_v, sem0, sem1, semp):
    cid = lax.axis_index("c")
    sid = lax.axis_index("s")
    wid = sid * _NC + cid
    base = wid * _BPW

    binit_v[pl.ds(1, 16)] = jnp.zeros((16,), jnp.float32)
    cw = pltpu.async_copy(w_hbm, w_v, semp)
    cb = pltpu.async_copy(b_hbm, binit_v.at[pl.ds(0, 1)], semp)
    cui = pltpu.async_copy(users_hbm.at[pl.ds(base, _BPW)], uidx_v, semp)
    cii = pltpu.async_copy(items_hbm.at[pl.ds(base, _BPW)], iidx_v, semp)
    cw.wait()
    cb.wait()
    cui.wait()
    cii.wait()
    b_init = binit_v[pl.ds(0, 16)]
    w_regs = [w_v[pl.ds(j * 16, 16)] for j in range(8)]

    lane = lax.iota(jnp.int32, 16)
    rot8 = (lane + 8) & 15
    rot4 = (lane & 8) | ((lane + 4) & 7)
    rot2 = (lane & 12) | ((lane + 2) & 3)
    rot1 = (lane & 14) | ((lane + 1) & 1)
    bitrev = (((lane & 1) << 3) | ((lane & 2) << 1)
              | ((lane & 4) >> 1) | ((lane & 8) >> 3))
    m8 = lane < 8
    m4 = (lane & 4) == 0
    m2 = (lane & 2) == 0
    m1 = (lane & 1) == 0

    _dnums = lax.GatherDimensionNumbers(
        offset_dims=(), collapsed_slice_dims=(0,), start_index_map=(0,))

    def _perm(x, idx):
        return lax.gather(x, idx[:, None], _dnums, (1,),
                          mode=lax.GatherScatterMode.PROMISE_IN_BOUNDS)

    ubufs = (ub0, ub1)
    ibufs = (ib0, ib1)
    sems = (sem0, sem1)
    pending = [None] * _NBUF

    _H = _CH // 2

    def start(c):
        k = c % _NBUF
        cps = []
        for h in range(2):
            cps.append(pltpu.async_copy(
                utab_hbm.at[uidx_v.at[pl.ds(c * _CH + h * _H, _H)]],
                ubufs[k].at[pl.ds(h * _H, _H)], sems[k]))
            cps.append(pltpu.async_copy(
                itab_hbm.at[iidx_v.at[pl.ds(c * _CH + h * _H, _H)]],
                ibufs[k].at[pl.ds(h * _H, _H)], sems[k]))
        pending[k] = cps

    for c in range(_NBUF - 1):
        start(c)

    for c in range(_NCHUNK):
        if c + _NBUF - 1 < _NCHUNK:
            start(c + _NBUF - 1)
        k = c % _NBUF
        for cp in pending[k]:
            cp.wait()
        urows_v = ubufs[k]
        irows_v = ibufs[k]

        def row_acc(r):
            acc = b_init
            for j in range(8):
                acc = acc + (urows_v[r, pl.ds(j * 16, 16)]
                             * irows_v[r, pl.ds(j * 16, 16)]
                             * w_regs[j])
            return acc

        def quad_body(i, carry):
            a = row_acc(4 * i)
            b = row_acc(4 * i + 1)
            cc = row_acc(4 * i + 2)
            d = row_acc(4 * i + 3)
            p1 = jnp.where(m8, a + _perm(a, rot8), b + _perm(b, rot8))
            p2 = jnp.where(m8, cc + _perm(cc, rot8), d + _perm(d, rot8))
            q1 = p1 + _perm(p1, rot4)
            q2 = p2 + _perm(p2, rot4)
            s = jnp.where(m4, q1, q2)
            qbuf_v[pl.ds(i * 16, 16)] = s + _perm(s, rot2)
            return carry

        lax.fori_loop(0, _CH // 4, quad_body, 0)

        def merge_body(g, carry):
            t0 = qbuf_v[pl.ds((4 * g) * 16, 16)]
            t1 = qbuf_v[pl.ds((4 * g + 1) * 16, 16)]
            t2 = qbuf_v[pl.ds((4 * g + 2) * 16, 16)]
            t3 = qbuf_v[pl.ds((4 * g + 3) * 16, 16)]
            u1 = jnp.where(m2, t0, t1)
            u2 = jnp.where(m2, t2, t3)
            v1 = u1 + _perm(u1, rot1)
            v2 = u2 + _perm(u2, rot1)
            f0 = jnp.where(m1, v1, v2)
            outc_v[pl.ds(c * _CH + g * 16, 16)] = _perm(f0, bitrev)
            return carry

        lax.fori_loop(0, _CH // 16, merge_body, 0)

    pltpu.sync_copy(outc_v, out_hbm.at[pl.ds(base, _BPW)])


def kernel(users, items, user_table, item_table, W_beta, b_beta):
    users_i = users.astype(jnp.int32)
    items_i = items.astype(jnp.int32)
    w = W_beta.reshape(_D)

    mesh = plsc.VectorSubcoreMesh(core_axis_name="c", subcore_axis_name="s")
    f = pl.kernel(
        _gmf_body,
        mesh=mesh,
        out_type=jax.ShapeDtypeStruct((_B,), jnp.float32),
        scratch_types=[
            pltpu.VMEM((_BPW,), jnp.int32),
            pltpu.VMEM((_BPW,), jnp.int32),
            pltpu.VMEM((_CH, _D), jnp.float32),
            pltpu.VMEM((_CH, _D), jnp.float32),
            pltpu.VMEM((_CH, _D), jnp.float32),
            pltpu.VMEM((_CH, _D), jnp.float32),
            pltpu.VMEM((_D,), jnp.float32),
            pltpu.VMEM((17,), jnp.float32),
            pltpu.VMEM((_CH * 4,), jnp.float32),
            pltpu.VMEM((_BPW,), jnp.float32),
            pltpu.SemaphoreType.DMA,
            pltpu.SemaphoreType.DMA,
            pltpu.SemaphoreType.DMA,
        ],
    )
    out = f(users_i, items_i, user_table, item_table, w, b_beta)
    return out.reshape(_B, 1)
